# baseline (speedup 1.0000x reference)
import jax
import jax.numpy as jnp
from jax.experimental import pallas as pl
from jax.experimental.pallas import tpu as pltpu


def _proj_body(u_ref, w_ref, b_ref, o_ref):
    acc = jnp.dot(u_ref[...], w_ref[...], preferred_element_type=jnp.float32)
    o_ref[...] = (acc + b_ref[...]).astype(o_ref.dtype)


def _project(ut, wx, bx, interpret=False):
    M, D = ut.shape
    N = wx.shape[1]
    bm = min(1024, M)
    grid = (M // bm,)
    return pl.pallas_call(
        _proj_body,
        out_shape=jax.ShapeDtypeStruct((M, N), jnp.bfloat16),
        grid=grid,
        in_specs=[
            pl.BlockSpec((bm, D), lambda i: (i, 0)),
            pl.BlockSpec((D, N), lambda i: (0, 0)),
            pl.BlockSpec((1, N), lambda i: (0, 0)),
        ],
        out_specs=pl.BlockSpec((bm, N), lambda i: (i, 0)),
        compiler_params=pltpu.CompilerParams(
            dimension_semantics=("parallel",),
            vmem_limit_bytes=48 * 1024 * 1024,
        ),
        name="speaker_rnn_project",
        interpret=interpret,
    )(ut, wx, bx)


def _make_rnn_body(H):
    def _rnn_body(spk_ref, xp_ref, wg_ref, wsg_ref, wsh_ref, wes_ref, bn_ref,
                  out_ref, g_ref, s0_ref, s1_ref):
        t = pl.program_id(0)

        @pl.when(t == 0)
        def _():
            g_ref[...] = jnp.zeros_like(g_ref)
            s0_ref[...] = jnp.zeros_like(s0_ref)
            s1_ref[...] = jnp.zeros_like(s1_ref)

        f32 = jnp.float32
        bf16 = jnp.bfloat16
        xp = xp_ref[...].astype(f32)
        g = g_ref[...]

        hh = jnp.dot(g.astype(bf16), wg_ref[...], preferred_element_type=f32)
        r = jax.nn.sigmoid(xp[:, :H] + hh[:, :H])
        z = jax.nn.sigmoid(xp[:, H:2 * H] + hh[:, H:2 * H])
        n = jnp.tanh(xp[:, 2 * H:3 * H] + r * (hh[:, 2 * H:] + bn_ref[0:1, :]))
        g_new = (1.0 - z) * n + z * g
        g_ref[...] = g_new

        m = jnp.transpose(spk_ref[0], (1, 0))
        s0 = s0_ref[...]
        s1 = s1_ref[...]
        s_prev = jnp.where(m < 0.5, s0, s1)
        sg = jnp.dot(g_new.astype(bf16), wsg_ref[...],
                     preferred_element_type=f32)
        sh = jnp.dot(s_prev.astype(bf16), wsh_ref[...],
                     preferred_element_type=f32)
        ps = xp[:, 3 * H:6 * H] + sg
        r_s = jax.nn.sigmoid(ps[:, :H] + sh[:, :H])
        z_s = jax.nn.sigmoid(ps[:, H:2 * H] + sh[:, H:2 * H])
        n_s = jnp.tanh(ps[:, 2 * H:] + r_s * (sh[:, 2 * H:] + bn_ref[1:2, :]))
        s_new = (1.0 - z_s) * n_s + z_s * s_prev
        s0_ref[...] = jnp.where(m < 0.5, s_new, s0)
        s1_ref[...] = jnp.where(m < 0.5, s1, s_new)

        es = jnp.dot(s_new.astype(bf16), wes_ref[...],
                     preferred_element_type=f32)
        pe = xp[:, 6 * H:] + es
        r_e = jax.nn.sigmoid(pe[:, :H])
        z_e = jax.nn.sigmoid(pe[:, H:2 * H])
        n_e = jnp.tanh(pe[:, 2 * H:] + r_e * bn_ref[2:3, :])
        out_ref[...] = (1.0 - z_e) * n_e

    return _rnn_body


def _forward(utt_embeds, speaker_ids,
             gW_ih, gW_hh, gb_ih, gb_hh,
             sW_ih, sW_hh, sb_ih, sb_hh,
             eW_ih, eW_hh, eb_ih, eb_hh,
             interpret=False):
    B, T, D = utt_embeds.shape
    H = gW_hh.shape[1]

    f32 = jnp.float32
    bf16 = jnp.bfloat16

    wx = jnp.concatenate([gW_ih, sW_ih[:, :D], eW_ih[:, :D]], axis=0).T

    def fold(b_ih, b_hh):
        return b_ih + jnp.concatenate([b_hh[:2 * H], jnp.zeros((H,), f32)])

    bx = jnp.concatenate(
        [fold(gb_ih, gb_hh), fold(sb_ih, sb_hh), fold(eb_ih, eb_hh)]
    ).reshape(1, 9 * H).astype(f32)

    ut = utt_embeds.reshape(B * T, D).astype(bf16)
    xp = _project(ut, wx.astype(bf16), bx, interpret=interpret)
    xp_slab = xp.reshape(B, T * 9 * H)

    spk = jnp.swapaxes(speaker_ids, 0, 1).astype(f32).reshape(T, 1, B)

    wg = gW_hh.T.astype(bf16)
    wsg = sW_ih[:, D:].T.astype(bf16)
    wsh = sW_hh.T.astype(bf16)
    wes = eW_ih[:, D:].T.astype(bf16)
    bn = jnp.stack([gb_hh[2 * H:], sb_hh[2 * H:], eb_hh[2 * H:]]).astype(f32)

    out = pl.pallas_call(
        _make_rnn_body(H),
        out_shape=jax.ShapeDtypeStruct((B, T * H), jnp.float32),
        grid=(T,),
        in_specs=[
            pl.BlockSpec((1, 1, B), lambda t: (t, 0, 0)),
            pl.BlockSpec((B, 9 * H), lambda t: (0, t)),
            pl.BlockSpec((H, 3 * H), lambda t: (0, 0)),
            pl.BlockSpec((H, 3 * H), lambda t: (0, 0)),
            pl.BlockSpec((H, 3 * H), lambda t: (0, 0)),
            pl.BlockSpec((H, 3 * H), lambda t: (0, 0)),
            pl.BlockSpec((3, H), lambda t: (0, 0)),
        ],
        out_specs=pl.BlockSpec((B, H), lambda t: (0, t)),
        scratch_shapes=[
            pltpu.VMEM((B, H), jnp.float32),
            pltpu.VMEM((B, H), jnp.float32),
            pltpu.VMEM((B, H), jnp.float32),
        ],
        compiler_params=pltpu.CompilerParams(
            dimension_semantics=("arbitrary",),
            vmem_limit_bytes=48 * 1024 * 1024,
        ),
        name="speaker_rnn_recurrence",
        interpret=interpret,
    )(spk, xp_slab, wg, wsg, wsh, wes, bn)

    return out.reshape(B, T, H)


def kernel(utt_embeds, speaker_ids,
           gW_ih, gW_hh, gb_ih, gb_hh,
           sW_ih, sW_hh, sb_ih, sb_hh,
           eW_ih, eW_hh, eb_ih, eb_hh):
    return _forward(utt_embeds, speaker_ids,
                    gW_ih, gW_hh, gb_ih, gb_hh,
                    sW_ih, sW_hh, sb_ih, sb_hh,
                    eW_ih, eW_hh, eb_ih, eb_hh)


# speedup vs baseline: 15.5350x; 15.5350x over previous
import jax
import jax.numpy as jnp
from jax.experimental import pallas as pl
from jax.experimental.pallas import tpu as pltpu


def _proj_body(u_ref, w_ref, b_ref, o_ref):
    acc = jnp.dot(u_ref[...], w_ref[...], preferred_element_type=jnp.float32)
    o_ref[...] = (acc + b_ref[...]).astype(o_ref.dtype)


def _project(ut, wx, bx, interpret=False):
    M, D = ut.shape
    N = wx.shape[1]
    bm = min(1024, M)
    grid = (M // bm,)
    return pl.pallas_call(
        _proj_body,
        out_shape=jax.ShapeDtypeStruct((M, N), jnp.bfloat16),
        grid=grid,
        in_specs=[
            pl.BlockSpec((bm, D), lambda i: (i, 0)),
            pl.BlockSpec((D, N), lambda i: (0, 0)),
            pl.BlockSpec((1, N), lambda i: (0, 0)),
        ],
        out_specs=pl.BlockSpec((bm, N), lambda i: (i, 0)),
        compiler_params=pltpu.CompilerParams(
            dimension_semantics=("parallel",),
            vmem_limit_bytes=48 * 1024 * 1024,
        ),
        name="speaker_rnn_project",
        interpret=interpret,
    )(ut, wx, bx)


def _make_rnn_body(H):
    def _rnn_body(spk_ref, xp_ref, wg_ref, wsg_ref, wsh_ref, wes_ref, bn_ref,
                  out_ref, g_ref, s0_ref, s1_ref):
        t = pl.program_id(0)

        @pl.when(t == 0)
        def _():
            g_ref[...] = jnp.zeros_like(g_ref)
            s0_ref[...] = jnp.zeros_like(s0_ref)
            s1_ref[...] = jnp.zeros_like(s1_ref)

        f32 = jnp.float32
        bf16 = jnp.bfloat16
        xp = xp_ref[...].astype(f32)
        g = g_ref[...]

        hh = jnp.dot(g.astype(bf16), wg_ref[...], preferred_element_type=f32)
        r = jax.nn.sigmoid(xp[:, :H] + hh[:, :H])
        z = jax.nn.sigmoid(xp[:, H:2 * H] + hh[:, H:2 * H])
        n = jnp.tanh(xp[:, 2 * H:3 * H] + r * (hh[:, 2 * H:] + bn_ref[0:1, :]))
        g_new = (1.0 - z) * n + z * g
        g_ref[...] = g_new

        m = jnp.transpose(spk_ref[0], (1, 0))
        s0 = s0_ref[...]
        s1 = s1_ref[...]
        s_prev = jnp.where(m < 0.5, s0, s1)
        sg = jnp.dot(g_new.astype(bf16), wsg_ref[...],
                     preferred_element_type=f32)
        sh = jnp.dot(s_prev.astype(bf16), wsh_ref[...],
                     preferred_element_type=f32)
        ps = xp[:, 3 * H:6 * H] + sg
        r_s = jax.nn.sigmoid(ps[:, :H] + sh[:, :H])
        z_s = jax.nn.sigmoid(ps[:, H:2 * H] + sh[:, H:2 * H])
        n_s = jnp.tanh(ps[:, 2 * H:] + r_s * (sh[:, 2 * H:] + bn_ref[1:2, :]))
        s_new = (1.0 - z_s) * n_s + z_s * s_prev
        s0_ref[...] = jnp.where(m < 0.5, s_new, s0)
        s1_ref[...] = jnp.where(m < 0.5, s1, s_new)

        es = jnp.dot(s_new.astype(bf16), wes_ref[...],
                     preferred_element_type=f32)
        pe = xp[:, 6 * H:] + es
        r_e = jax.nn.sigmoid(pe[:, :H])
        z_e = jax.nn.sigmoid(pe[:, H:2 * H])
        n_e = jnp.tanh(pe[:, 2 * H:] + r_e * bn_ref[2:3, :])
        out_ref[...] = (1.0 - z_e) * n_e

    return _rnn_body


def _forward(utt_embeds, speaker_ids,
             gW_ih, gW_hh, gb_ih, gb_hh,
             sW_ih, sW_hh, sb_ih, sb_hh,
             eW_ih, eW_hh, eb_ih, eb_hh,
             interpret=False):
    B, T, D = utt_embeds.shape
    H = gW_hh.shape[1]

    f32 = jnp.float32
    bf16 = jnp.bfloat16

    wx = jnp.concatenate([gW_ih, sW_ih[:, :D], eW_ih[:, :D]], axis=0).T

    def fold(b_ih, b_hh):
        return b_ih + jnp.concatenate([b_hh[:2 * H], jnp.zeros((H,), f32)])

    bx = jnp.concatenate(
        [fold(gb_ih, gb_hh), fold(sb_ih, sb_hh), fold(eb_ih, eb_hh)]
    ).reshape(1, 9 * H).astype(f32)

    ut = jnp.swapaxes(utt_embeds, 0, 1).reshape(T * B, D).astype(bf16)
    xp = _project(ut, wx.astype(bf16), bx, interpret=interpret)

    spk = jnp.swapaxes(speaker_ids, 0, 1).astype(f32).reshape(T, 1, B)

    wg = gW_hh.T.astype(bf16)
    wsg = sW_ih[:, D:].T.astype(bf16)
    wsh = sW_hh.T.astype(bf16)
    wes = eW_ih[:, D:].T.astype(bf16)
    bn = jnp.stack([gb_hh[2 * H:], sb_hh[2 * H:], eb_hh[2 * H:]]).astype(f32)

    out = pl.pallas_call(
        _make_rnn_body(H),
        out_shape=jax.ShapeDtypeStruct((T * B, H), jnp.float32),
        grid=(T,),
        in_specs=[
            pl.BlockSpec((1, 1, B), lambda t: (t, 0, 0)),
            pl.BlockSpec((B, 9 * H), lambda t: (t, 0)),
            pl.BlockSpec((H, 3 * H), lambda t: (0, 0)),
            pl.BlockSpec((H, 3 * H), lambda t: (0, 0)),
            pl.BlockSpec((H, 3 * H), lambda t: (0, 0)),
            pl.BlockSpec((H, 3 * H), lambda t: (0, 0)),
            pl.BlockSpec((3, H), lambda t: (0, 0)),
        ],
        out_specs=pl.BlockSpec((B, H), lambda t: (t, 0)),
        scratch_shapes=[
            pltpu.VMEM((B, H), jnp.float32),
            pltpu.VMEM((B, H), jnp.float32),
            pltpu.VMEM((B, H), jnp.float32),
        ],
        compiler_params=pltpu.CompilerParams(
            dimension_semantics=("arbitrary",),
            vmem_limit_bytes=48 * 1024 * 1024,
        ),
        name="speaker_rnn_recurrence",
        interpret=interpret,
    )(spk, xp, wg, wsg, wsh, wes, bn)

    return jnp.swapaxes(out.reshape(T, B, H), 0, 1)


def kernel(utt_embeds, speaker_ids,
           gW_ih, gW_hh, gb_ih, gb_hh,
           sW_ih, sW_hh, sb_ih, sb_hh,
           eW_ih, eW_hh, eb_ih, eb_hh):
    return _forward(utt_embeds, speaker_ids,
                    gW_ih, gW_hh, gb_ih, gb_hh,
                    sW_ih, sW_hh, sb_ih, sb_hh,
                    eW_ih, eW_hh, eb_ih, eb_hh)


# speedup vs baseline: 15.9385x; 1.0260x over previous
import jax
import jax.numpy as jnp
from jax.experimental import pallas as pl
from jax.experimental.pallas import tpu as pltpu


def _proj_body(u_ref, w_ref, b_ref, o_ref):
    acc = jnp.dot(u_ref[...], w_ref[...], preferred_element_type=jnp.float32)
    o_ref[...] = (acc + b_ref[...]).astype(o_ref.dtype)


def _project(ut, wx, bx, interpret=False):
    M, D = ut.shape
    N = wx.shape[1]
    bm = min(1024, M)
    grid = (M // bm,)
    return pl.pallas_call(
        _proj_body,
        out_shape=jax.ShapeDtypeStruct((M, N), jnp.bfloat16),
        grid=grid,
        in_specs=[
            pl.BlockSpec((bm, D), lambda i: (i, 0)),
            pl.BlockSpec((D, N), lambda i: (0, 0)),
            pl.BlockSpec((1, N), lambda i: (0, 0)),
        ],
        out_specs=pl.BlockSpec((bm, N), lambda i: (i, 0)),
        compiler_params=pltpu.CompilerParams(
            dimension_semantics=("parallel",),
            vmem_limit_bytes=48 * 1024 * 1024,
        ),
        name="speaker_rnn_project",
        interpret=interpret,
    )(ut, wx, bx)


def _sig(x):
    return 0.5 + 0.5 * jnp.tanh(0.5 * x)


def _make_rnn_body(H, unroll):
    f32 = jnp.float32
    bf16 = jnp.bfloat16

    def _rnn_body(spk_ref, xp_ref, wg_ref, wsg_ref, wsh_ref, wes_ref, bn_ref,
                  out_ref, g_ref, s0_ref, s1_ref):
        t = pl.program_id(0)

        @pl.when(t == 0)
        def _():
            g_ref[...] = jnp.zeros_like(g_ref)
            s0_ref[...] = jnp.zeros_like(s0_ref)
            s1_ref[...] = jnp.zeros_like(s1_ref)

        B = g_ref.shape[0]
        for u in range(unroll):
            xp = xp_ref[u * B:(u + 1) * B, :].astype(f32)
            g = g_ref[...]

            hh = jnp.dot(g.astype(bf16), wg_ref[...],
                         preferred_element_type=f32)
            r = _sig(xp[:, :H] + hh[:, :H])
            z = _sig(xp[:, H:2 * H] + hh[:, H:2 * H])
            n = jnp.tanh(xp[:, 2 * H:3 * H]
                         + r * (hh[:, 2 * H:] + bn_ref[0:1, :]))
            g_new = (1.0 - z) * n + z * g
            g_ref[...] = g_new

            m = jnp.transpose(spk_ref[0, u:u + 1, :], (1, 0))
            s0 = s0_ref[...]
            s1 = s1_ref[...]
            s_prev = jnp.where(m < 0.5, s0, s1)
            sg = jnp.dot(g_new.astype(bf16), wsg_ref[...],
                         preferred_element_type=f32)
            sh = jnp.dot(s_prev.astype(bf16), wsh_ref[...],
                         preferred_element_type=f32)
            ps = xp[:, 3 * H:6 * H] + sg
            r_s = _sig(ps[:, :H] + sh[:, :H])
            z_s = _sig(ps[:, H:2 * H] + sh[:, H:2 * H])
            n_s = jnp.tanh(ps[:, 2 * H:]
                           + r_s * (sh[:, 2 * H:] + bn_ref[1:2, :]))
            s_new = (1.0 - z_s) * n_s + z_s * s_prev
            s0_ref[...] = jnp.where(m < 0.5, s_new, s0)
            s1_ref[...] = jnp.where(m < 0.5, s1, s_new)

            es = jnp.dot(s_new.astype(bf16), wes_ref[...],
                         preferred_element_type=f32)
            pe = xp[:, 6 * H:] + es
            r_e = _sig(pe[:, :H])
            z_e = _sig(pe[:, H:2 * H])
            n_e = jnp.tanh(pe[:, 2 * H:] + r_e * bn_ref[2:3, :])
            out_ref[u * B:(u + 1) * B, :] = (1.0 - z_e) * n_e

    return _rnn_body


def _forward(utt_embeds, speaker_ids,
             gW_ih, gW_hh, gb_ih, gb_hh,
             sW_ih, sW_hh, sb_ih, sb_hh,
             eW_ih, eW_hh, eb_ih, eb_hh,
             interpret=False):
    B, T, D = utt_embeds.shape
    H = gW_hh.shape[1]

    f32 = jnp.float32
    bf16 = jnp.bfloat16

    wx = jnp.concatenate([gW_ih, sW_ih[:, :D], eW_ih[:, :D]], axis=0).T

    def fold(b_ih, b_hh):
        return b_ih + jnp.concatenate([b_hh[:2 * H], jnp.zeros((H,), f32)])

    bx = jnp.concatenate(
        [fold(gb_ih, gb_hh), fold(sb_ih, sb_hh), fold(eb_ih, eb_hh)]
    ).reshape(1, 9 * H).astype(f32)

    ut = jnp.swapaxes(utt_embeds.astype(bf16), 0, 1).reshape(T * B, D)
    xp = _project(ut, wx.astype(bf16), bx, interpret=interpret)

    UNROLL = 2
    spk = jnp.swapaxes(speaker_ids, 0, 1).astype(f32).reshape(
        T // UNROLL, UNROLL, B)

    wg = gW_hh.T.astype(bf16)
    wsg = sW_ih[:, D:].T.astype(bf16)
    wsh = sW_hh.T.astype(bf16)
    wes = eW_ih[:, D:].T.astype(bf16)
    bn = jnp.stack([gb_hh[2 * H:], sb_hh[2 * H:], eb_hh[2 * H:]]).astype(f32)

    out = pl.pallas_call(
        _make_rnn_body(H, UNROLL),
        out_shape=jax.ShapeDtypeStruct((T * B, H), jnp.float32),
        grid=(T // UNROLL,),
        in_specs=[
            pl.BlockSpec((1, UNROLL, B), lambda t: (t, 0, 0)),
            pl.BlockSpec((UNROLL * B, 9 * H), lambda t: (t, 0)),
            pl.BlockSpec((H, 3 * H), lambda t: (0, 0)),
            pl.BlockSpec((H, 3 * H), lambda t: (0, 0)),
            pl.BlockSpec((H, 3 * H), lambda t: (0, 0)),
            pl.BlockSpec((H, 3 * H), lambda t: (0, 0)),
            pl.BlockSpec((3, H), lambda t: (0, 0)),
        ],
        out_specs=pl.BlockSpec((UNROLL * B, H), lambda t: (t, 0)),
        scratch_shapes=[
            pltpu.VMEM((B, H), jnp.float32),
            pltpu.VMEM((B, H), jnp.float32),
            pltpu.VMEM((B, H), jnp.float32),
        ],
        compiler_params=pltpu.CompilerParams(
            dimension_semantics=("arbitrary",),
            vmem_limit_bytes=48 * 1024 * 1024,
        ),
        name="speaker_rnn_recurrence",
        interpret=interpret,
    )(spk, xp, wg, wsg, wsh, wes, bn)

    return jnp.swapaxes(out.reshape(T, B, H), 0, 1)


def kernel(utt_embeds, speaker_ids,
           gW_ih, gW_hh, gb_ih, gb_hh,
           sW_ih, sW_hh, sb_ih, sb_hh,
           eW_ih, eW_hh, eb_ih, eb_hh):
    return _forward(utt_embeds, speaker_ids,
                    gW_ih, gW_hh, gb_ih, gb_hh,
                    sW_ih, sW_hh, sb_ih, sb_hh,
                    eW_ih, eW_hh, eb_ih, eb_hh)


# speedup vs baseline: 17.1929x; 1.0787x over previous
import jax
import jax.numpy as jnp
from jax.experimental import pallas as pl
from jax.experimental.pallas import tpu as pltpu


def _proj_body(u_ref, w_ref, b_ref, o_ref):
    acc = jnp.dot(u_ref[...], w_ref[...], preferred_element_type=jnp.float32)
    o_ref[...] = (acc + b_ref[...]).astype(o_ref.dtype)


def _project(ut, wx, bx, interpret=False):
    M, D = ut.shape
    N = wx.shape[1]
    bm = min(1024, M)
    grid = (M // bm,)
    return pl.pallas_call(
        _proj_body,
        out_shape=jax.ShapeDtypeStruct((M, N), jnp.bfloat16),
        grid=grid,
        in_specs=[
            pl.BlockSpec((bm, D), lambda i: (i, 0)),
            pl.BlockSpec((D, N), lambda i: (0, 0)),
            pl.BlockSpec((1, N), lambda i: (0, 0)),
        ],
        out_specs=pl.BlockSpec((bm, N), lambda i: (i, 0)),
        compiler_params=pltpu.CompilerParams(
            dimension_semantics=("parallel",),
            vmem_limit_bytes=48 * 1024 * 1024,
        ),
        name="speaker_rnn_project",
        interpret=interpret,
    )(ut, wx, bx)


def _sig(x):
    return 0.5 + 0.5 * jnp.tanh(0.5 * x)


def _make_rnn_body(H, unroll):
    f32 = jnp.float32
    bf16 = jnp.bfloat16

    def _rnn_body(spk_ref, xp_ref, wg_ref, wsg_ref, wsh_ref, wes_ref, bn_ref,
                  out_ref, g_ref, s0_ref, s1_ref):
        t = pl.program_id(0)

        @pl.when(t == 0)
        def _():
            g_ref[...] = jnp.zeros_like(g_ref)
            s0_ref[...] = jnp.zeros_like(s0_ref)
            s1_ref[...] = jnp.zeros_like(s1_ref)

        B = g_ref.shape[0]
        s_news = []
        pe_xs = []
        for u in range(unroll):
            xp = xp_ref[u * B:(u + 1) * B, :].astype(f32)
            g = g_ref[...]

            hh = jnp.dot(g.astype(bf16), wg_ref[...],
                         preferred_element_type=f32)
            r = _sig(xp[:, :H] + hh[:, :H])
            z = _sig(xp[:, H:2 * H] + hh[:, H:2 * H])
            n = jnp.tanh(xp[:, 2 * H:3 * H]
                         + r * (hh[:, 2 * H:] + bn_ref[0:1, :]))
            g_new = (1.0 - z) * n + z * g
            g_ref[...] = g_new

            m = jnp.transpose(spk_ref[0, u:u + 1, :], (1, 0))
            s0 = s0_ref[...]
            s1 = s1_ref[...]
            s_prev = jnp.where(m < 0.5, s0, s1)
            sg = jnp.dot(g_new.astype(bf16), wsg_ref[...],
                         preferred_element_type=f32)
            sh = jnp.dot(s_prev.astype(bf16), wsh_ref[...],
                         preferred_element_type=f32)
            ps = xp[:, 3 * H:6 * H] + sg
            r_s = _sig(ps[:, :H] + sh[:, :H])
            z_s = _sig(ps[:, H:2 * H] + sh[:, H:2 * H])
            n_s = jnp.tanh(ps[:, 2 * H:]
                           + r_s * (sh[:, 2 * H:] + bn_ref[1:2, :]))
            s_new = (1.0 - z_s) * n_s + z_s * s_prev
            s0_ref[...] = jnp.where(m < 0.5, s_new, s0)
            s1_ref[...] = jnp.where(m < 0.5, s1, s_new)
            s_news.append(s_new.astype(bf16))
            pe_xs.append(xp[:, 6 * H:])

        s_cat = jnp.concatenate(s_news, axis=0)
        es = jnp.dot(s_cat, wes_ref[...], preferred_element_type=f32)
        pe = jnp.concatenate(pe_xs, axis=0) + es
        r_e = _sig(pe[:, :H])
        z_e = _sig(pe[:, H:2 * H])
        n_e = jnp.tanh(pe[:, 2 * H:] + r_e * bn_ref[2:3, :])
        out_ref[...] = (1.0 - z_e) * n_e

    return _rnn_body


def _forward(utt_embeds, speaker_ids,
             gW_ih, gW_hh, gb_ih, gb_hh,
             sW_ih, sW_hh, sb_ih, sb_hh,
             eW_ih, eW_hh, eb_ih, eb_hh,
             interpret=False):
    B, T, D = utt_embeds.shape
    H = gW_hh.shape[1]

    f32 = jnp.float32
    bf16 = jnp.bfloat16

    wx = jnp.concatenate([gW_ih, sW_ih[:, :D], eW_ih[:, :D]], axis=0).T

    def fold(b_ih, b_hh):
        return b_ih + jnp.concatenate([b_hh[:2 * H], jnp.zeros((H,), f32)])

    bx = jnp.concatenate(
        [fold(gb_ih, gb_hh), fold(sb_ih, sb_hh), fold(eb_ih, eb_hh)]
    ).reshape(1, 9 * H).astype(f32)

    ut = jnp.swapaxes(utt_embeds.astype(bf16), 0, 1).reshape(T * B, D)
    xp = _project(ut, wx.astype(bf16), bx, interpret=interpret)

    UNROLL = 4
    spk = jnp.swapaxes(speaker_ids, 0, 1).astype(f32).reshape(
        T // UNROLL, UNROLL, B)

    wg = gW_hh.T.astype(bf16)
    wsg = sW_ih[:, D:].T.astype(bf16)
    wsh = sW_hh.T.astype(bf16)
    wes = eW_ih[:, D:].T.astype(bf16)
    bn = jnp.stack([gb_hh[2 * H:], sb_hh[2 * H:], eb_hh[2 * H:]]).astype(f32)

    out = pl.pallas_call(
        _make_rnn_body(H, UNROLL),
        out_shape=jax.ShapeDtypeStruct((T * B, H), jnp.float32),
        grid=(T // UNROLL,),
        in_specs=[
            pl.BlockSpec((1, UNROLL, B), lambda t: (t, 0, 0)),
            pl.BlockSpec((UNROLL * B, 9 * H), lambda t: (t, 0)),
            pl.BlockSpec((H, 3 * H), lambda t: (0, 0)),
            pl.BlockSpec((H, 3 * H), lambda t: (0, 0)),
            pl.BlockSpec((H, 3 * H), lambda t: (0, 0)),
            pl.BlockSpec((H, 3 * H), lambda t: (0, 0)),
            pl.BlockSpec((3, H), lambda t: (0, 0)),
        ],
        out_specs=pl.BlockSpec((UNROLL * B, H), lambda t: (t, 0)),
        scratch_shapes=[
            pltpu.VMEM((B, H), jnp.float32),
            pltpu.VMEM((B, H), jnp.float32),
            pltpu.VMEM((B, H), jnp.float32),
        ],
        compiler_params=pltpu.CompilerParams(
            dimension_semantics=("arbitrary",),
            vmem_limit_bytes=48 * 1024 * 1024,
        ),
        name="speaker_rnn_recurrence",
        interpret=interpret,
    )(spk, xp, wg, wsg, wsh, wes, bn)

    return jnp.swapaxes(out.reshape(T, B, H), 0, 1)


def kernel(utt_embeds, speaker_ids,
           gW_ih, gW_hh, gb_ih, gb_hh,
           sW_ih, sW_hh, sb_ih, sb_hh,
           eW_ih, eW_hh, eb_ih, eb_hh):
    return _forward(utt_embeds, speaker_ids,
                    gW_ih, gW_hh, gb_ih, gb_hh,
                    sW_ih, sW_hh, sb_ih, sb_hh,
                    eW_ih, eW_hh, eb_ih, eb_hh)
